# baseline (speedup 1.0000x reference)
import functools

import jax
import jax.numpy as jnp
from jax import lax
from jax.experimental import pallas as pl
from jax.experimental.pallas import tpu as pltpu
from jax.experimental.pallas import tpu_sc as plsc

_INFO = plsc.get_sparse_core_info()
_NC = _INFO.num_cores
_NS = _INFO.num_subcores
_NW = _NC * _NS
_LANE = 128
_W65 = 80


def _compact_fn(vocab, hidden):
    mesh = plsc.VectorSubcoreMesh(core_axis_name="c", subcore_axis_name="s")
    nvt = (vocab + _LANE - 1) // _LANE
    vp = nvt * _LANE
    kmax = (nvt + _NW - 1) // _NW

    @functools.partial(
        pl.kernel,
        mesh=mesh,
        out_type=jax.ShapeDtypeStruct((vp * _W65,), jnp.float32),
        scratch_types=[
            pltpu.VMEM((2, hidden, _LANE), jnp.float32),
            pltpu.VMEM((2 * _LANE * _W65,), jnp.float32),
            pltpu.VMEM((hidden * 129,), jnp.float32),
            pltpu.SemaphoreType.DMA((2,)),
            pltpu.SemaphoreType.DMA,
        ],
        compiler_params=pltpu.CompilerParams(
            use_tc_tiling_on_sc=True, needs_layout_passes=False),
    )
    def k(wt_hbm, out_hbm, inb, sb, sk, rsem, wsem):
        wid = lax.axis_index("s") * _NC + lax.axis_index("c")
        lanes = lax.iota(jnp.int32, 16)
        lanes129 = lanes * 129
        sblk = _LANE * _W65

        def rstart(vt, b2):
            pltpu.async_copy(
                wt_hbm.at[:, pl.ds(vt * _LANE, _LANE)], inb.at[b2],
                rsem.at[b2])

        def rwait(vt, b2):
            pltpu.make_async_copy(
                wt_hbm.at[:, pl.ds(vt * _LANE, _LANE)], inb.at[b2],
                rsem.at[b2]).wait()

        def wdrain():
            pltpu.make_async_copy(
                out_hbm.at[pl.ds(0, sblk)],
                sb.at[pl.ds(0, sblk)], wsem).wait()

        def transpose(b2, boff):
            def hloop(h, carry):
                for c8 in range(8):
                    v = inb[b2, h, pl.ds(16 * c8, 16)]
                    plsc.store_scatter(sk, [lanes + (h * 129 + 16 * c8)], v)
                return carry
            lax.fori_loop(0, hidden, hloop, 0)

            def vloop(v0, carry):
                for h0 in range(hidden // 16):
                    idx = lanes129 + (129 * 16 * h0 + v0)
                    vv = plsc.load_gather(sk, [idx])
                    sb[pl.ds(boff + v0 * _W65 + 16 * h0, 16)] = vv
                return carry
            lax.fori_loop(0, _LANE, vloop, 0)

        rstart(wid, 0)

        def body(kk, carry):
            vt = wid + _NW * kk
            b2 = lax.rem(kk, 2)

            @pl.when(vt < nvt)
            def _():
                rwait(vt, b2)

                @pl.when(vt + _NW < nvt)
                def _():
                    rstart(vt + _NW, 1 - b2)

                @pl.when(kk >= 2)
                def _():
                    wdrain()

                boff = b2 * sblk
                transpose(b2, boff)
                pltpu.async_copy(
                    sb.at[pl.ds(boff, sblk)],
                    out_hbm.at[pl.ds(vt * sblk, sblk)], wsem)

            return carry

        lax.fori_loop(0, kmax, body, 0)
        wdrain()
        wdrain()

    return k


def _gather_fn(batch, seq, hidden, vp):
    mesh = plsc.VectorSubcoreMesh(core_axis_name="c", subcore_axis_name="s")
    h8 = hidden // 8
    assert batch == _NW * _LANE and seq % 2 == 0

    @functools.partial(
        pl.kernel,
        mesh=mesh,
        out_type=jax.ShapeDtypeStruct((seq, h8, _NW * 1024), jnp.float32),
        scratch_types=[
            pltpu.VMEM((seq, _LANE), jnp.int32),
            pltpu.VMEM((_LANE, _W65), jnp.float32),
            pltpu.VMEM((_LANE, _W65), jnp.float32),
            pltpu.VMEM((h8, 1024), jnp.float32),
            pltpu.VMEM((h8, 1024), jnp.float32),
            pltpu.VMEM((_LANE * 65,), jnp.float32),
            pltpu.SemaphoreType.DMA((2,)),
            pltpu.SemaphoreType.DMA((2,)),
        ],
        compiler_params=pltpu.CompilerParams(
            use_tc_tiling_on_sc=False, needs_layout_passes=False),
    )
    def k(ids_hbm, tbl_hbm, out_hbm, idx_v, g0, g1, t0, t1, s2, gsem, ssem):
        wid = lax.axis_index("s") * _NC + lax.axis_index("c")
        gbuf = (g0, g1)
        tbuf = (t0, t1)
        pltpu.sync_copy(ids_hbm.at[:, pl.ds(wid * _LANE, _LANE)], idx_v)

        def gather_start(s, b):
            pltpu.async_copy(tbl_hbm.at[idx_v.at[s]], gbuf[b], gsem.at[b])

        def gather_wait(s, b):
            pltpu.make_async_copy(
                tbl_hbm.at[idx_v.at[s]], gbuf[b], gsem.at[b]).wait()

        def store_start(s, b):
            pltpu.async_copy(
                tbuf[b], out_hbm.at[s, :, pl.ds(wid * 1024, 1024)], ssem.at[b])

        def store_wait(s, b):
            pltpu.make_async_copy(
                tbuf[b], out_hbm.at[s, :, pl.ds(wid * 1024, 1024)],
                ssem.at[b]).wait()

        lanes = lax.iota(jnp.int32, 16)
        lanes65 = lanes * 65

        def transpose(b):
            g, t = gbuf[b], tbuf[b]

            def aloop(r, carry):
                for c4 in range(hidden // 16):
                    v = g[r, pl.ds(16 * c4, 16)]
                    plsc.store_scatter(s2, [lanes + (r * 65 + 16 * c4)], v)
                return carry
            lax.fori_loop(0, _LANE, aloop, 0)

            def qloop(q, carry):
                for hr in range(8):
                    h = q * 8 + hr
                    for bb in range(8):
                        idx = lanes65 + (bb * 16 * 65 + h)
                        v = plsc.load_gather(s2, [idx])
                        t[q, pl.ds(hr * 128 + bb * 16, 16)] = v
                return carry
            lax.fori_loop(0, h8, qloop, 0)

        gather_start(0, 0)
        gather_start(1, 1)
        for b in range(2):
            gather_wait(b, b)
            transpose(b)
            gather_start(b + 2, b)
            store_start(b, b)

        def main(g, carry):
            s0 = 2 * g
            for b in range(2):
                s = s0 + b
                gather_wait(s, b)
                store_wait(s - 2, b)
                transpose(b)
                gather_start(s + 2, b)
                store_start(s, b)
            return carry

        lax.fori_loop(1, seq // 2 - 1, main, 0)

        for b in range(2):
            s = seq - 2 + b
            gather_wait(s, b)
            store_wait(s - 2, b)
            transpose(b)
            store_start(s, b)
        for b in range(2):
            store_wait(seq - 2 + b, b)

    return k


def _retile_fn(batch, seq, hidden):
    mesh = plsc.VectorSubcoreMesh(core_axis_name="c", subcore_axis_name="s")
    n_rows = seq * (hidden // 8)
    nbt = batch // _LANE
    rows_per_w = n_rows // _NW
    assert n_rows % _NW == 0 and rows_per_w % 2 == 0

    @functools.partial(
        pl.kernel,
        mesh=mesh,
        out_type=jax.ShapeDtypeStruct((seq, hidden, batch), jnp.float32),
        scratch_types=[
            pltpu.VMEM((nbt, 8, _LANE), jnp.float32),
            pltpu.VMEM((nbt, 8, _LANE), jnp.float32),
            pltpu.SemaphoreType.DMA((2,)),
            pltpu.SemaphoreType.DMA((2,)),
        ],
        compiler_params=pltpu.CompilerParams(use_tc_tiling_on_sc=True),
    )
    def k(in_hbm, out_hbm, b0, b1, rsem, wsem):
        wid = lax.axis_index("s") * _NC + lax.axis_index("c")
        y0 = wid * rows_per_w
        bufs = (b0, b1)

        def rstart(y, b):
            pltpu.async_copy(in_hbm.at[pl.ds(y * nbt, nbt)], bufs[b], rsem.at[b])

        def rwait(y, b):
            pltpu.make_async_copy(
                in_hbm.at[pl.ds(y * nbt, nbt)], bufs[b], rsem.at[b]).wait()

        def wstart(y, b):
            s = y // (hidden // 8)
            hr = y % (hidden // 8)
            for i in range(nbt):
                pltpu.async_copy(
                    bufs[b].at[i],
                    out_hbm.at[s, pl.ds(8 * hr, 8), pl.ds(_LANE * i, _LANE)],
                    wsem.at[b])

        def wwait(b):
            pltpu.make_async_copy(
                in_hbm.at[pl.ds(0, nbt)], bufs[b], wsem.at[b]).wait()

        rstart(y0, 0)
        rstart(y0 + 1, 1)

        def main(i, carry):
            for b in range(2):
                y = y0 + 2 * i + b
                rwait(y, b)
                wstart(y, b)
                wwait(b)
                pl.when(2 * i + b + 2 < rows_per_w)(
                    lambda yb=y, bb=b: rstart(yb + 2, bb))
            return carry

        lax.fori_loop(0, rows_per_w // 2, main, 0)

    return k


def kernel(input_ids, word_embeddings):
    batch, seq = input_ids.shape
    vocab, hidden = word_embeddings.shape
    assert batch == _NW * _LANE
    nvt = (vocab + _LANE - 1) // _LANE
    vp = nvt * _LANE

    ids_t = input_ids.T.astype(jnp.int32)
    w_t = word_embeddings.T
    tbl = _compact_fn(vocab, hidden)(w_t)
    o = _gather_fn(batch, seq, hidden, vp)(ids_t, tbl.reshape(vp, _W65))
    n_tiles = seq * (hidden // 8) * (batch // _LANE)
    tiles = o.reshape(n_tiles, 8, _LANE)
    o3 = _retile_fn(batch, seq, hidden)(tiles)
    return o3.transpose(2, 0, 1)


# speedup vs baseline: 1.5504x; 1.5504x over previous
import functools

import jax
import jax.numpy as jnp
from jax import lax
from jax.experimental import pallas as pl
from jax.experimental.pallas import tpu as pltpu
from jax.experimental.pallas import tpu_sc as plsc

_INFO = plsc.get_sparse_core_info()
_NC = _INFO.num_cores
_NS = _INFO.num_subcores
_NW = _NC * _NS

_NBUF = 4


def _gather_fn(batch, seq, hidden):
    mesh = plsc.VectorSubcoreMesh(core_axis_name="c", subcore_axis_name="s")
    rows_per_w = batch // _NW
    n_main = rows_per_w - _NBUF
    assert n_main >= 0 and n_main % _NBUF == 0

    @functools.partial(
        pl.kernel,
        mesh=mesh,
        out_type=jax.ShapeDtypeStruct((batch, seq, hidden), jnp.float32),
        scratch_types=[
            pltpu.VMEM((rows_per_w, seq), jnp.int32),
            pltpu.VMEM((_NBUF, seq, hidden), jnp.float32),
            pltpu.SemaphoreType.DMA((_NBUF,)),
            pltpu.SemaphoreType.DMA((_NBUF,)),
        ],
        compiler_params=pltpu.CompilerParams(use_tc_tiling_on_sc=False),
    )
    def k(idx_hbm, table_hbm, out_hbm, idx_v, rows_v, gsem, ssem):
        wid = lax.axis_index("s") * _NC + lax.axis_index("c")
        base = wid * rows_per_w
        pltpu.sync_copy(idx_hbm.at[pl.ds(base, rows_per_w)], idx_v)

        def gather_start(j, b):
            pltpu.async_copy(table_hbm.at[idx_v.at[j]], rows_v.at[b], gsem.at[b])

        def gather_wait(j, b):
            pltpu.make_async_copy(
                table_hbm.at[idx_v.at[j]], rows_v.at[b], gsem.at[b]).wait()

        def store_start(j, b):
            pltpu.async_copy(rows_v.at[b], out_hbm.at[base + j], ssem.at[b])

        def store_wait(j, b):
            pltpu.make_async_copy(
                rows_v.at[b], out_hbm.at[base + j], ssem.at[b]).wait()

        for b in range(_NBUF):
            gather_start(b, b)

        def outer(g, carry):
            j0 = g * _NBUF
            for b in range(_NBUF):
                j = j0 + b
                gather_wait(j, b)
                store_start(j, b)
            for b in range(_NBUF):
                j = j0 + b
                store_wait(j, b)
                gather_start(j + _NBUF, b)
            return carry

        lax.fori_loop(0, n_main // _NBUF, outer, 0)

        for b in range(_NBUF):
            j = n_main + b
            gather_wait(j, b)
            store_start(j, b)
        for b in range(_NBUF):
            store_wait(n_main + b, b)

    return k


def kernel(input_ids, word_embeddings):
    batch, seq = input_ids.shape
    vocab, hidden = word_embeddings.shape
    assert batch % _NW == 0
    ids = input_ids.astype(jnp.int32)
    return _gather_fn(batch, seq, hidden)(ids, word_embeddings)
